# baseline (speedup 1.0000x reference)
import functools

import jax
import jax.numpy as jnp
from jax.experimental import pallas as pl
from jax.experimental.pallas import tpu as pltpu

S, B, D, F = 4096, 2, 1024, 4096
BM = 512
BR = BM // B
FK = 1024


def _fused_ffn_kernel(x_ref, c_ref, g_ref, b_ref, w1_ref, b1_ref, w2_ref,
                      b2_ref, o_ref):
    x = x_ref[...].reshape(BM, D)

    mu = jnp.mean(x, axis=1, keepdims=True)
    xc = x - mu
    var = jnp.mean(xc * xc, axis=1, keepdims=True)
    h = xc * jax.lax.rsqrt(var + 1e-5) * g_ref[...] + b_ref[...]

    logit = jnp.sum(x * c_ref[...], axis=1, keepdims=True)
    alpha = jax.nn.sigmoid(logit)

    hb = h.astype(jnp.bfloat16)
    nk = F // FK
    h2 = b2_ref[...] * jnp.ones((BM, 1), jnp.float32)
    for k in range(nk):
        h1 = jax.lax.dot_general(
            hb, w1_ref[k * FK:(k + 1) * FK, :],
            dimension_numbers=(((1,), (1,)), ((), ())),
            preferred_element_type=jnp.float32)
        a = jnp.maximum(h1 + b1_ref[:, k * FK:(k + 1) * FK], 0.0)
        h2 = h2 + jax.lax.dot_general(
            a.astype(jnp.bfloat16), w2_ref[:, k * FK:(k + 1) * FK],
            dimension_numbers=(((1,), (1,)), ((), ())),
            preferred_element_type=jnp.float32)

    o_ref[...] = (x + alpha * h2).reshape(BR, B, D)


def _cast_kernel(w1_ref, w2_ref, o1_ref, o2_ref):
    o1_ref[...] = w1_ref[...].astype(jnp.bfloat16)
    o2_ref[...] = w2_ref[...].astype(jnp.bfloat16)


def _cast_weights(w1, w2):
    n = 8
    return pl.pallas_call(
        _cast_kernel,
        grid=(n,),
        in_specs=[
            pl.BlockSpec((F // n, D), lambda i: (i, 0)),
            pl.BlockSpec((D // n, F), lambda i: (i, 0)),
        ],
        out_specs=[
            pl.BlockSpec((F // n, D), lambda i: (i, 0)),
            pl.BlockSpec((D // n, F), lambda i: (i, 0)),
        ],
        out_shape=[
            jax.ShapeDtypeStruct((F, D), jnp.bfloat16),
            jax.ShapeDtypeStruct((D, F), jnp.bfloat16),
        ],
        compiler_params=pltpu.CompilerParams(
            dimension_semantics=("arbitrary",),
        ),
    )(w1, w2)


@jax.jit
def _run(x, c, g, b, w1, b1, w2, b2):
    w1, w2 = _cast_weights(w1, w2)
    grid = (S // BR,)
    const = lambda shape: pl.BlockSpec(shape, lambda i: (0, 0))
    return pl.pallas_call(
        _fused_ffn_kernel,
        grid=grid,
        in_specs=[
            pl.BlockSpec((BR, B, D), lambda i: (i, 0, 0)),
            const((1, D)),
            const((1, D)),
            const((1, D)),
            const((F, D)),
            const((1, F)),
            const((D, F)),
            const((1, D)),
        ],
        out_specs=pl.BlockSpec((BR, B, D), lambda i: (i, 0, 0)),
        out_shape=jax.ShapeDtypeStruct((S, B, D), jnp.float32),
        compiler_params=pltpu.CompilerParams(
            dimension_semantics=("arbitrary",),
        ),
    )(x, c, g, b, w1, b1, w2, b2)


def kernel(input_features, expert_centroids, ln_g, ln_b, ff1_w, ff1_b, ff2_w,
           ff2_b):
    return _run(
        input_features,
        expert_centroids.reshape(1, D),
        ln_g.reshape(1, D),
        ln_b.reshape(1, D),
        ff1_w,
        ff1_b.reshape(1, F),
        ff2_w,
        ff2_b.reshape(1, D),
    )


# speedup vs baseline: 1.0087x; 1.0087x over previous
import functools

import jax
import jax.numpy as jnp
from jax.experimental import pallas as pl
from jax.experimental.pallas import tpu as pltpu

S, B, D, F = 4096, 2, 1024, 4096
BM = 1024
BR = BM // B
FK = 1024


def _fused_ffn_kernel(x_ref, c_ref, g_ref, b_ref, w1_ref, b1_ref, w2_ref,
                      b2_ref, o_ref):
    x = x_ref[...].reshape(BM, D)

    mu = jnp.mean(x, axis=1, keepdims=True)
    xc = x - mu
    var = jnp.mean(xc * xc, axis=1, keepdims=True)
    h = xc * jax.lax.rsqrt(var + 1e-5) * g_ref[...] + b_ref[...]

    logit = jnp.sum(x * c_ref[...], axis=1, keepdims=True)
    alpha = jax.nn.sigmoid(logit)

    hb = h.astype(jnp.bfloat16)
    nk = F // FK
    h2 = b2_ref[...] * jnp.ones((BM, 1), jnp.float32)
    for k in range(nk):
        h1 = jax.lax.dot_general(
            hb, w1_ref[k * FK:(k + 1) * FK, :],
            dimension_numbers=(((1,), (1,)), ((), ())),
            preferred_element_type=jnp.float32)
        a = jnp.maximum(h1 + b1_ref[:, k * FK:(k + 1) * FK], 0.0)
        h2 = h2 + jax.lax.dot_general(
            a.astype(jnp.bfloat16), w2_ref[:, k * FK:(k + 1) * FK],
            dimension_numbers=(((1,), (1,)), ((), ())),
            preferred_element_type=jnp.float32)

    o_ref[...] = (x + alpha * h2).reshape(BR, B, D)


def _cast_kernel(w1_ref, w2_ref, o1_ref, o2_ref):
    o1_ref[...] = w1_ref[...].astype(jnp.bfloat16)
    o2_ref[...] = w2_ref[...].astype(jnp.bfloat16)


def _cast_weights(w1, w2):
    n = 8
    return pl.pallas_call(
        _cast_kernel,
        grid=(n,),
        in_specs=[
            pl.BlockSpec((F // n, D), lambda i: (i, 0)),
            pl.BlockSpec((D // n, F), lambda i: (i, 0)),
        ],
        out_specs=[
            pl.BlockSpec((F // n, D), lambda i: (i, 0)),
            pl.BlockSpec((D // n, F), lambda i: (i, 0)),
        ],
        out_shape=[
            jax.ShapeDtypeStruct((F, D), jnp.bfloat16),
            jax.ShapeDtypeStruct((D, F), jnp.bfloat16),
        ],
        compiler_params=pltpu.CompilerParams(
            dimension_semantics=("arbitrary",),
        ),
    )(w1, w2)


@jax.jit
def _run(x, c, g, b, w1, b1, w2, b2):
    w1, w2 = _cast_weights(w1, w2)
    grid = (S // BR,)
    const = lambda shape: pl.BlockSpec(shape, lambda i: (0, 0))
    return pl.pallas_call(
        _fused_ffn_kernel,
        grid=grid,
        in_specs=[
            pl.BlockSpec((BR, B, D), lambda i: (i, 0, 0)),
            const((1, D)),
            const((1, D)),
            const((1, D)),
            const((F, D)),
            const((1, F)),
            const((D, F)),
            const((1, D)),
        ],
        out_specs=pl.BlockSpec((BR, B, D), lambda i: (i, 0, 0)),
        out_shape=jax.ShapeDtypeStruct((S, B, D), jnp.float32),
        compiler_params=pltpu.CompilerParams(
            dimension_semantics=("arbitrary",),
        ),
    )(x, c, g, b, w1, b1, w2, b2)


def kernel(input_features, expert_centroids, ln_g, ln_b, ff1_w, ff1_b, ff2_w,
           ff2_b):
    return _run(
        input_features,
        expert_centroids.reshape(1, D),
        ln_g.reshape(1, D),
        ln_b.reshape(1, D),
        ff1_w,
        ff1_b.reshape(1, F),
        ff2_w,
        ff2_b.reshape(1, D),
    )
